# baseline (speedup 1.0000x reference)
import numpy as np

import jax
import jax.numpy as jnp
from jax.experimental import pallas as pl
from jax.experimental.pallas import tpu as pltpu

_HO = 13
_G = 14
_FP = _G * _G
_XR = 210
_MO = _HO * _G
_MP = 184
_PB = 16

_TAP = {0: (0, 0), 1: (1, 0), 2: (0, 1)}


def _mask_np():
    m = np.zeros((2, 2, _FP, 192), np.float32)
    for u in range(2):
        for v in range(2):
            pm = 14 if u == 0 else 13
            qm = 14 if v == 0 else 13
            m2 = np.zeros((_G, _G), np.float32)
            m2[:pm, :qm] = 1.0
            m[u, v] = np.broadcast_to(m2.reshape(_FP, 1), (_FP, 192))
    return m


_MASK = _mask_np()


def _mega_kernel(x_ref, mask_ref, w0_ref, w1_ref, w2_ref, w3_ref,
                 b_ref, o_ref, p_ref, y2_ref, a0_ref):

    zhead = jnp.zeros((_PB, 192), jnp.bfloat16)
    ztail = jnp.zeros((240 - _PB - _FP, 192), jnp.bfloat16)
    b1 = b_ref[1, :, 0:192]
    for u in range(2):
        for v in range(2):
            y = jnp.dot(x_ref[0, u, v, 0:_FP, :], w1_ref[...],
                        preferred_element_type=jnp.float32)
            y = jnp.maximum(y + b1, 0.0)
            p_ref[u, v, _PB:_PB + _FP, :] = (
                y.astype(jnp.bfloat16) * mask_ref[u, v])
            p_ref[u, v, 0:_PB, :] = zhead
            p_ref[u, v, _PB + _FP:, :] = ztail

    b2 = b_ref[2, :, 0:224]
    for u in range(2):
        for v in range(2):
            acc = jnp.zeros((_FP, 224), jnp.float32)
            for dh in range(3):
                e = u + dh - 1
                up, pa = e % 2, (e - (e % 2)) // 2
                for dw in range(3):
                    f = v + dw - 1
                    vp, qa = f % 2, (f - (f % 2)) // 2
                    ofs = _PB + _G * pa + qa
                    acc = acc + jnp.dot(p_ref[up, vp, ofs:ofs + _FP, :],
                                        w2_ref[dh * 3 + dw],
                                        preferred_element_type=jnp.float32)
            y2 = jnp.maximum(acc + b2, 0.0)
            y2_ref[u, v, 0:_FP, :] = y2.astype(jnp.bfloat16)
            y2_ref[u, v, _FP:, :] = jnp.zeros((224 - _FP, 224), jnp.bfloat16)

    b3 = b_ref[3, :, 0:256]
    acc1 = jnp.zeros((_MO, 256), jnp.float32)
    for dh in range(3):
        up, pa = _TAP[dh]
        for dw in range(3):
            vp, qa = _TAP[dw]
            ofs = _G * pa + qa
            acc1 = acc1 + jnp.dot(y2_ref[up, vp, ofs:ofs + _MO, :],
                                  w3_ref[dh * 3 + dw],
                                  preferred_element_type=jnp.float32)
    x1 = jnp.maximum(acc1 + b3, 0.0)

    b0 = b_ref[0, :, 0:384]
    acc0 = jnp.zeros((_MO, 384), jnp.float32)
    for dh in range(3):
        up, pa = _TAP[dh]
        for dw in range(3):
            vp, qa = _TAP[dw]
            ofs = _G * pa + qa
            acc0 = acc0 + jnp.dot(x_ref[0, up, vp, ofs:ofs + _MO, :],
                                  w0_ref[dh * 3 + dw],
                                  preferred_element_type=jnp.float32)
    x0 = jnp.maximum(acc0 + b0, 0.0)

    for v in range(2):
        av = jnp.maximum(
            jnp.maximum(x_ref[0, 0, v, 0:_MP, :], x_ref[0, 1, v, 0:_MP, :]),
            x_ref[0, 0, v, _G:_G + _MP, :])
        if v == 0:
            a0_ref[...] = av
            a0v = av
        else:
            a1v = av
    mx = jnp.maximum(jnp.maximum(a0v[0:_MO, :], a1v[0:_MO, :]),
                     a0_ref[1:1 + _MO, :])

    o_ref[0, :, 0:384] = x0
    o_ref[0, :, 384:640] = x1
    o_ref[0, :, 640:1024] = mx.astype(jnp.float32)


def kernel(x, branch0_wk, branch0_b, branch1_0_wk, branch1_0_b,
           branch1_1_wk, branch1_1_b, branch1_2_wk, branch1_2_b):
    N = x.shape[0]
    xp = jnp.pad(x.astype(jnp.bfloat16), ((0, 0), (0, 0), (0, 3), (0, 1)))
    xp = xp.reshape(N, 384, _XR // _G, 2, _G, 2)
    xs2d = jnp.transpose(xp, (0, 3, 5, 2, 4, 1)).reshape(N, 2, 2, _XR, 384)
    mask = jnp.asarray(_MASK, jnp.bfloat16)
    bias = jnp.stack([
        branch0_b,
        jnp.pad(branch1_0_b, (0, 192)),
        jnp.pad(branch1_1_b, (0, 160)),
        jnp.pad(branch1_2_b, (0, 128)),
    ]).reshape(4, 1, 384)

    out = pl.pallas_call(
        _mega_kernel,
        out_shape=jax.ShapeDtypeStruct((N, _MO, 1024), jnp.float32),
        grid_spec=pltpu.PrefetchScalarGridSpec(
            num_scalar_prefetch=0,
            grid=(N,),
            in_specs=[
                pl.BlockSpec((1, 2, 2, _XR, 384), lambda n: (n, 0, 0, 0, 0)),
                pl.BlockSpec((2, 2, _FP, 192), lambda n: (0, 0, 0, 0)),
                pl.BlockSpec((9, 384, 384), lambda n: (0, 0, 0)),
                pl.BlockSpec((384, 192), lambda n: (0, 0)),
                pl.BlockSpec((9, 192, 224), lambda n: (0, 0, 0)),
                pl.BlockSpec((9, 224, 256), lambda n: (0, 0, 0)),
                pl.BlockSpec((4, 1, 384), lambda n: (0, 0, 0)),
            ],
            out_specs=pl.BlockSpec((1, _MO, 1024), lambda n: (n, 0, 0)),
            scratch_shapes=[
                pltpu.VMEM((2, 2, 240, 192), jnp.bfloat16),
                pltpu.VMEM((2, 2, 224, 224), jnp.bfloat16),
                pltpu.VMEM((_MP, 384), jnp.bfloat16),
            ],
        ),
        compiler_params=pltpu.CompilerParams(
            dimension_semantics=("parallel",)),
    )(xs2d, mask, branch0_wk, branch1_0_wk, branch1_1_wk,
      branch1_2_wk, bias)

    out = out.reshape(N, _HO, _G, 1024)[:, :, :_HO, :]
    return jnp.transpose(out, (0, 3, 1, 2))
